# baseline (speedup 1.0000x reference)
import functools

import jax
import jax.numpy as jnp
from jax import lax
from jax.experimental import pallas as pl
from jax.experimental.pallas import tpu as pltpu
from jax.experimental.pallas import tpu_sc as plsc

B = 16384
D = 32
NC = 2
NS = 16
L = 16
NW = NC * NS
BPW = B // NW
NSLOT = 8
ROUNDS = BPW // NSLOT


def _sc_body(uid_hbm, iid_hbm, nid_hbm, ut_hbm, it_hbm,
             score_hbm, nscore_hbm,
             uids_v, iids_v, nids_v,
             ubufs, ibufs, nbufs, score_v, nscore_v, sems):
    wid = lax.axis_index("s") * NC + lax.axis_index("c")
    base = wid * BPW
    iota = lax.iota(jnp.int32, L)
    rows_lo = iota
    rows_hi = iota + L

    def ids_at(idx):
        win = pl.multiple_of(lax.shift_left(lax.shift_right_logical(idx, 4), 4), L)
        mask = iota == jnp.bitwise_and(idx, L - 1)
        zero = jnp.zeros((L,), jnp.int32)
        u = jnp.sum(jnp.where(mask, uids_v[pl.ds(win, L)], zero))
        i = jnp.sum(jnp.where(mask, iids_v[pl.ds(win, L)], zero))
        n = jnp.sum(jnp.where(mask, nids_v[pl.ds(win, L)], zero))
        return u, i, n

    def fire(k, idx):
        u, i, n = ids_at(idx)
        uoff = pl.multiple_of(lax.shift_left(lax.shift_right_logical(u, 7), 7), 128)
        ioff = pl.multiple_of(lax.shift_left(lax.shift_right_logical(i, 7), 7), 128)
        noff = pl.multiple_of(lax.shift_left(lax.shift_right_logical(n, 7), 7), 128)
        for a in range(D // 8):
            rs = pl.ds(a * 8, 8)
            pltpu.async_copy(ut_hbm.at[rs, pl.ds(uoff, 128)],
                             ubufs.at[k, rs], sems.at[k])
            pltpu.async_copy(it_hbm.at[rs, pl.ds(ioff, 128)],
                             ibufs.at[k, rs], sems.at[k])
            pltpu.async_copy(it_hbm.at[rs, pl.ds(noff, 128)],
                             nbufs.at[k, rs], sems.at[k])

    def drain(k):
        for a in range(D // 8):
            rs = pl.ds(a * 8, 8)
            pltpu.make_async_copy(ut_hbm.at[rs, pl.ds(0, 128)],
                                  ubufs.at[k, rs], sems.at[k]).wait()
            pltpu.make_async_copy(ut_hbm.at[rs, pl.ds(0, 128)],
                                  ibufs.at[k, rs], sems.at[k]).wait()
            pltpu.make_async_copy(ut_hbm.at[rs, pl.ds(0, 128)],
                                  nbufs.at[k, rs], sems.at[k]).wait()

    def extract(k, idx, acc_s, acc_n):
        u, i, n = ids_at(idx)
        ulane = jnp.full((L,), jnp.bitwise_and(u, 127), jnp.int32)
        ilane = jnp.full((L,), jnp.bitwise_and(i, 127), jnp.int32)
        nlane = jnp.full((L,), jnp.bitwise_and(n, 127), jnp.int32)
        u0 = plsc.load_gather(ubufs.at[k], [rows_lo, ulane])
        u1 = plsc.load_gather(ubufs.at[k], [rows_hi, ulane])
        i0 = plsc.load_gather(ibufs.at[k], [rows_lo, ilane])
        i1 = plsc.load_gather(ibufs.at[k], [rows_hi, ilane])
        n0 = plsc.load_gather(nbufs.at[k], [rows_lo, nlane])
        n1 = plsc.load_gather(nbufs.at[k], [rows_hi, nlane])
        s = jnp.sum(u0 * i0 + u1 * i1)
        t = jnp.sum(u0 * n0 + u1 * n1)
        mask = iota == jnp.bitwise_and(idx, L - 1)
        return (jnp.where(mask, jnp.full((L,), s, jnp.float32), acc_s),
                jnp.where(mask, jnp.full((L,), t, jnp.float32), acc_n))

    pltpu.sync_copy(uid_hbm.at[pl.ds(base, BPW)], uids_v)
    pltpu.sync_copy(iid_hbm.at[pl.ds(base, BPW)], iids_v)
    pltpu.sync_copy(nid_hbm.at[pl.ds(base, BPW)], nids_v)

    for k in range(NSLOT):
        fire(k, k)

    zeros = jnp.zeros((L,), jnp.float32)

    def round_body(r, carry):
        acc_s, acc_n = carry
        for k in range(NSLOT):
            idx = r * NSLOT + k
            drain(k)
            acc_s, acc_n = extract(k, idx, acc_s, acc_n)

            @pl.when(r < ROUNDS - 1)
            def _():
                fire(k, idx + NSLOT)

        @pl.when(jnp.bitwise_and(r, 1) == 1)
        def _():
            off = pl.multiple_of((r - 1) * NSLOT, L)
            score_v[pl.ds(off, L)] = acc_s
            nscore_v[pl.ds(off, L)] = acc_n

        odd = jnp.bitwise_and(r, 1) == 1
        return (jnp.where(odd, zeros, acc_s), jnp.where(odd, zeros, acc_n))

    lax.fori_loop(0, ROUNDS, round_body, (zeros, zeros))

    pltpu.sync_copy(score_v, score_hbm.at[pl.ds(base, BPW)])
    pltpu.sync_copy(nscore_v, nscore_hbm.at[pl.ds(base, BPW)])


def kernel(user_id, item_id, neg_item_id, user_memory, item_memory):
    mesh = plsc.VectorSubcoreMesh(core_axis_name="c", subcore_axis_name="s")
    run = functools.partial(
        pl.kernel,
        mesh=mesh,
        out_type=(jax.ShapeDtypeStruct((B,), jnp.float32),
                  jax.ShapeDtypeStruct((B,), jnp.float32)),
        scratch_types=[
            pltpu.VMEM((BPW,), jnp.int32),
            pltpu.VMEM((BPW,), jnp.int32),
            pltpu.VMEM((BPW,), jnp.int32),
            pltpu.VMEM((NSLOT, D, 128), jnp.float32),
            pltpu.VMEM((NSLOT, D, 128), jnp.float32),
            pltpu.VMEM((NSLOT, D, 128), jnp.float32),
            pltpu.VMEM((BPW,), jnp.float32),
            pltpu.VMEM((BPW,), jnp.float32),
            pltpu.SemaphoreType.DMA((NSLOT,)),
        ],
        compiler_params=pltpu.CompilerParams(needs_layout_passes=False,
                                             disable_bounds_checks=True),
    )(_sc_body)
    return run(user_id.astype(jnp.int32), item_id.astype(jnp.int32),
               neg_item_id.astype(jnp.int32),
               user_memory.T, item_memory.T)


# speedup vs baseline: 1.0094x; 1.0094x over previous
import functools

import jax
import jax.numpy as jnp
from jax import lax
from jax.experimental import pallas as pl
from jax.experimental.pallas import tpu as pltpu
from jax.experimental.pallas import tpu_sc as plsc

B = 16384
D = 32
NC = 2
NS = 16
L = 16
NW = NC * NS
BPW = B // NW
NSLOT = 8
ROUNDS = BPW // NSLOT


def _sc_body(uid_hbm, iid_hbm, nid_hbm, ut_hbm, it_hbm,
             score_hbm, nscore_hbm,
             uids_v, iids_v, nids_v,
             ubufs, ibufs, nbufs, score_v, nscore_v, sems):
    wid = lax.axis_index("s") * NC + lax.axis_index("c")
    base = wid * BPW
    iota = lax.iota(jnp.int32, L)
    rows_lo = iota
    rows_hi = iota + L

    def ids_at(idx):
        win = pl.multiple_of(lax.shift_left(lax.shift_right_logical(idx, 4), 4), L)
        mask = iota == jnp.bitwise_and(idx, L - 1)
        zero = jnp.zeros((L,), jnp.int32)
        u = jnp.sum(jnp.where(mask, uids_v[pl.ds(win, L)], zero))
        i = jnp.sum(jnp.where(mask, iids_v[pl.ds(win, L)], zero))
        n = jnp.sum(jnp.where(mask, nids_v[pl.ds(win, L)], zero))
        return u, i, n

    def fire(k, idx):
        u, i, n = ids_at(idx)
        uoff = pl.multiple_of(lax.shift_left(lax.shift_right_logical(u, 7), 7), 128)
        ioff = pl.multiple_of(lax.shift_left(lax.shift_right_logical(i, 7), 7), 128)
        noff = pl.multiple_of(lax.shift_left(lax.shift_right_logical(n, 7), 7), 128)
        pltpu.async_copy(ut_hbm.at[:, pl.ds(uoff, 128)], ubufs.at[k], sems.at[k])
        pltpu.async_copy(it_hbm.at[:, pl.ds(ioff, 128)], ibufs.at[k], sems.at[k])
        pltpu.async_copy(it_hbm.at[:, pl.ds(noff, 128)], nbufs.at[k], sems.at[k])

    def drain(k):
        pltpu.make_async_copy(ut_hbm.at[:, pl.ds(0, 128)], ubufs.at[k], sems.at[k]).wait()
        pltpu.make_async_copy(ut_hbm.at[:, pl.ds(0, 128)], ibufs.at[k], sems.at[k]).wait()
        pltpu.make_async_copy(ut_hbm.at[:, pl.ds(0, 128)], nbufs.at[k], sems.at[k]).wait()

    def extract(k, idx, acc_s, acc_n):
        u, i, n = ids_at(idx)
        ulane = jnp.full((L,), jnp.bitwise_and(u, 127), jnp.int32)
        ilane = jnp.full((L,), jnp.bitwise_and(i, 127), jnp.int32)
        nlane = jnp.full((L,), jnp.bitwise_and(n, 127), jnp.int32)
        u0 = plsc.load_gather(ubufs.at[k], [rows_lo, ulane])
        u1 = plsc.load_gather(ubufs.at[k], [rows_hi, ulane])
        i0 = plsc.load_gather(ibufs.at[k], [rows_lo, ilane])
        i1 = plsc.load_gather(ibufs.at[k], [rows_hi, ilane])
        n0 = plsc.load_gather(nbufs.at[k], [rows_lo, nlane])
        n1 = plsc.load_gather(nbufs.at[k], [rows_hi, nlane])
        s = jnp.sum(u0 * i0 + u1 * i1)
        t = jnp.sum(u0 * n0 + u1 * n1)
        mask = iota == jnp.bitwise_and(idx, L - 1)
        return (jnp.where(mask, jnp.full((L,), s, jnp.float32), acc_s),
                jnp.where(mask, jnp.full((L,), t, jnp.float32), acc_n))

    pltpu.sync_copy(uid_hbm.at[pl.ds(base, BPW)], uids_v)
    pltpu.sync_copy(iid_hbm.at[pl.ds(base, BPW)], iids_v)
    pltpu.sync_copy(nid_hbm.at[pl.ds(base, BPW)], nids_v)

    for k in range(NSLOT):
        fire(k, k)

    zeros = jnp.zeros((L,), jnp.float32)

    def round_body(r, carry):
        acc_s, acc_n = carry
        for k in range(NSLOT):
            idx = r * NSLOT + k
            drain(k)
            acc_s, acc_n = extract(k, idx, acc_s, acc_n)

            @pl.when(r < ROUNDS - 1)
            def _():
                fire(k, idx + NSLOT)

        @pl.when(jnp.bitwise_and(r, 1) == 1)
        def _():
            off = pl.multiple_of((r - 1) * NSLOT, L)
            score_v[pl.ds(off, L)] = acc_s
            nscore_v[pl.ds(off, L)] = acc_n

        odd = jnp.bitwise_and(r, 1) == 1
        return (jnp.where(odd, zeros, acc_s), jnp.where(odd, zeros, acc_n))

    lax.fori_loop(0, ROUNDS, round_body, (zeros, zeros))

    pltpu.sync_copy(score_v, score_hbm.at[pl.ds(base, BPW)])
    pltpu.sync_copy(nscore_v, nscore_hbm.at[pl.ds(base, BPW)])


def kernel(user_id, item_id, neg_item_id, user_memory, item_memory):
    mesh = plsc.VectorSubcoreMesh(core_axis_name="c", subcore_axis_name="s")
    run = functools.partial(
        pl.kernel,
        mesh=mesh,
        out_type=(jax.ShapeDtypeStruct((B,), jnp.float32),
                  jax.ShapeDtypeStruct((B,), jnp.float32)),
        scratch_types=[
            pltpu.VMEM((BPW,), jnp.int32),
            pltpu.VMEM((BPW,), jnp.int32),
            pltpu.VMEM((BPW,), jnp.int32),
            pltpu.VMEM((NSLOT, D, 128), jnp.float32),
            pltpu.VMEM((NSLOT, D, 128), jnp.float32),
            pltpu.VMEM((NSLOT, D, 128), jnp.float32),
            pltpu.VMEM((BPW,), jnp.float32),
            pltpu.VMEM((BPW,), jnp.float32),
            pltpu.SemaphoreType.DMA((NSLOT,)),
        ],
        compiler_params=pltpu.CompilerParams(needs_layout_passes=False,
                                             disable_bounds_checks=True),
    )(_sc_body)
    return run(user_id.astype(jnp.int32), item_id.astype(jnp.int32),
               neg_item_id.astype(jnp.int32),
               user_memory.T, item_memory.T)
